# Initial kernel scaffold; baseline (speedup 1.0000x reference)
import jax
import jax.numpy as jnp
from jax.experimental import pallas as pl


def kernel(batch_data, item_embeddings):
    raise NotImplementedError("write your pallas kernel here")


# speedup vs baseline: 1.6952x; 1.6952x over previous
import functools

import jax
import jax.numpy as jnp
from jax import lax
from jax.experimental import pallas as pl
from jax.experimental.pallas import tpu as pltpu
from jax.experimental.pallas import tpu_sc as plsc

NC = 2
NS = 16
NW = NC * NS

B, S = 16384, 50
D = 64
ROWS = B * S
CHUNK = 128
NCHUNKS = ROWS // CHUNK
CHUNKS_PER_W = NCHUNKS // NW
ROWS_PER_W = ROWS // NW


def kernel(batch_data, item_embeddings):
    idx2d = batch_data.reshape(NCHUNKS, CHUNK)
    mesh = plsc.VectorSubcoreMesh(core_axis_name="c", subcore_axis_name="s")

    @functools.partial(
        pl.kernel,
        mesh=mesh,
        compiler_params=pltpu.CompilerParams(use_tc_tiling_on_sc=False),
        out_type=jax.ShapeDtypeStruct((ROWS, D), jnp.float32),
        scratch_types=[
            pltpu.VMEM((CHUNKS_PER_W, CHUNK), jnp.int32),
            pltpu.VMEM((CHUNK, D), jnp.float32),
            pltpu.SemaphoreType.DMA,
        ],
    )
    def gather_kernel(idx_hbm, table_hbm, out_hbm, idx_v, rows_v, gsem):
        wid = lax.axis_index("s") * NC + lax.axis_index("c")
        chunk_base = wid * CHUNKS_PER_W
        row_base = wid * ROWS_PER_W
        pltpu.sync_copy(idx_hbm.at[pl.ds(chunk_base, CHUNKS_PER_W)], idx_v)

        def body(j, carry):
            pltpu.async_copy(table_hbm.at[idx_v.at[j]], rows_v, gsem).wait()
            pltpu.sync_copy(
                rows_v, out_hbm.at[pl.ds(row_base + j * CHUNK, CHUNK)]
            )
            return carry

        lax.fori_loop(0, CHUNKS_PER_W, body, 0)

    out = gather_kernel(idx2d, item_embeddings)
    return out.reshape(B, S, D)


# speedup vs baseline: 1.8644x; 1.0998x over previous
import functools

import jax
import jax.numpy as jnp
from jax import lax
from jax.experimental import pallas as pl
from jax.experimental.pallas import tpu as pltpu
from jax.experimental.pallas import tpu_sc as plsc

NC = 2
NS = 16
NW = NC * NS

B, S = 16384, 50
D = 64
ROWS = B * S
CHUNK = 128
NCHUNKS = ROWS // CHUNK
N = NCHUNKS // NW
ROWS_PER_W = ROWS // NW
NBUF = 4


def kernel(batch_data, item_embeddings):
    idx2d = batch_data.reshape(NCHUNKS, CHUNK)
    mesh = plsc.VectorSubcoreMesh(core_axis_name="c", subcore_axis_name="s")

    @functools.partial(
        pl.kernel,
        mesh=mesh,
        compiler_params=pltpu.CompilerParams(use_tc_tiling_on_sc=False),
        out_type=jax.ShapeDtypeStruct((ROWS, D), jnp.float32),
        scratch_types=[
            pltpu.VMEM((N, CHUNK), jnp.int32),
            pltpu.VMEM((NBUF, CHUNK, D), jnp.float32),
        ] + [pltpu.SemaphoreType.DMA] * (2 * NBUF),
    )
    def gather_kernel(idx_hbm, table_hbm, out_hbm, idx_v, rows_v, *sems):
        gsem = sems[:NBUF]
        ssem = sems[NBUF:]
        wid = lax.axis_index("s") * NC + lax.axis_index("c")
        chunk_base = wid * N
        row_base = wid * ROWS_PER_W
        pltpu.sync_copy(idx_hbm.at[pl.ds(chunk_base, N)], idx_v)

        def gather_copy(j, b):
            return pltpu.make_async_copy(
                table_hbm.at[idx_v.at[j]], rows_v.at[b], gsem[b]
            )

        def write_copy(j, b):
            return pltpu.make_async_copy(
                rows_v.at[b],
                out_hbm.at[pl.ds(row_base + j * CHUNK, CHUNK)],
                ssem[b],
            )

        gather_copy(0, 0).start()
        gather_copy(1, 1).start()

        def body(g, carry):
            for b in range(NBUF):
                j = NBUF * g + b
                gather_copy(j, b).wait()
                write_copy(j, b).start()
                b2 = (b + 2) % NBUF
                if b >= 2:
                    write_copy(j - 2, b2).wait()
                else:
                    @pl.when(g > 0)
                    def _():
                        write_copy(j - 2, b2).wait()

                @pl.when(j + 2 < N)
                def _():
                    gather_copy(j + 2, b2).start()
            return carry

        lax.fori_loop(0, N // NBUF, body, 0)
        write_copy(N - 2, (N - 2) % NBUF).wait()
        write_copy(N - 1, (N - 1) % NBUF).wait()

    out = gather_kernel(idx2d, item_embeddings)
    return out.reshape(B, S, D)


# speedup vs baseline: 1.8753x; 1.0059x over previous
import functools

import jax
import jax.numpy as jnp
from jax import lax
from jax.experimental import pallas as pl
from jax.experimental.pallas import tpu as pltpu
from jax.experimental.pallas import tpu_sc as plsc

NC = 2
NS = 16
NW = NC * NS

B, S = 16384, 50
D = 64
ROWS = B * S
CHUNK = 128
NCHUNKS = ROWS // CHUNK
N = NCHUNKS // NW
ROWS_PER_W = ROWS // NW
NBUF = 8
LEAD = NBUF // 2


def kernel(batch_data, item_embeddings):
    idx2d = batch_data.reshape(NCHUNKS, CHUNK)
    mesh = plsc.VectorSubcoreMesh(core_axis_name="c", subcore_axis_name="s")

    @functools.partial(
        pl.kernel,
        mesh=mesh,
        compiler_params=pltpu.CompilerParams(use_tc_tiling_on_sc=False),
        out_type=jax.ShapeDtypeStruct((ROWS, D), jnp.float32),
        scratch_types=[
            pltpu.VMEM((N, CHUNK), jnp.int32),
            pltpu.VMEM((NBUF, CHUNK, D), jnp.float32),
        ] + [pltpu.SemaphoreType.DMA] * (2 * NBUF),
    )
    def gather_kernel(idx_hbm, table_hbm, out_hbm, idx_v, rows_v, *sems):
        gsem = sems[:NBUF]
        ssem = sems[NBUF:]
        wid = lax.axis_index("s") * NC + lax.axis_index("c")
        chunk_base = wid * N
        row_base = wid * ROWS_PER_W
        pltpu.sync_copy(idx_hbm.at[pl.ds(chunk_base, N)], idx_v)

        def gather_copy(j, b):
            return pltpu.make_async_copy(
                table_hbm.at[idx_v.at[j]], rows_v.at[b], gsem[b]
            )

        def write_copy(j, b):
            return pltpu.make_async_copy(
                rows_v.at[b],
                out_hbm.at[pl.ds(row_base + j * CHUNK, CHUNK)],
                ssem[b],
            )

        for j0 in range(LEAD):
            gather_copy(j0, j0 % NBUF).start()

        def body(g, carry):
            for b in range(NBUF):
                j = NBUF * g + b
                gather_copy(j, b).wait()
                write_copy(j, b).start()
                b2 = (b + LEAD) % NBUF
                if b >= LEAD:
                    write_copy(j - LEAD, b2).wait()
                else:
                    @pl.when(g > 0)
                    def _():
                        write_copy(j - LEAD, b2).wait()

                @pl.when(j + LEAD < N)
                def _():
                    gather_copy(j + LEAD, b2).start()
            return carry

        lax.fori_loop(0, N // NBUF, body, 0)
        for j0 in range(N - LEAD, N):
            write_copy(j0, j0 % NBUF).wait()

    out = gather_kernel(idx2d, item_embeddings)
    return out.reshape(B, S, D)
